# baseline (speedup 1.0000x reference)
import functools

import jax
import jax.numpy as jnp
from jax import lax
from jax.experimental import pallas as pl
from jax.experimental.pallas import tpu as pltpu
from jax.experimental.pallas import tpu_sc as plsc

N_NODES = 100000
N_EDGES = 3200000
NODE_DIM = 128
EDGE_DIM = 16
NUM_GRAPHS = 64
LATENT = 128

NW = 32
E_PER_W = N_EDGES // NW
N_PER_W = N_NODES // NW
N_PAD_W = 3200
EB = 2000
NB = 400
E_BLOCKS = E_PER_W // EB
N_BLOCKS = N_PAD_W // NB
N_LAST = N_PER_W - (N_BLOCKS - 1) * NB

_SEG_F = NUM_GRAPHS * NODE_DIM
_CNT_F = NUM_GRAPHS * 16
_ACCN_F = (NUM_GRAPHS + 1) * NODE_DIM
_ACCC_F = (NUM_GRAPHS + 1) * 16


@functools.partial(
    pl.kernel,
    mesh=plsc.VectorSubcoreMesh(core_axis_name="c", subcore_axis_name="s"),
    compiler_params=pltpu.CompilerParams(needs_layout_passes=False),
    out_type=[
        jax.ShapeDtypeStruct((NW, _SEG_F), jnp.float32),
        jax.ShapeDtypeStruct((NW, _CNT_F), jnp.float32),
        jax.ShapeDtypeStruct((NW, 128), jnp.float32),
    ],
    scratch_types=[
        pltpu.VMEM((EB * EDGE_DIM,), jnp.float32),
        pltpu.VMEM((NB * NODE_DIM,), jnp.float32),
        pltpu.VMEM((N_PAD_W,), jnp.int32),
        pltpu.VMEM((_ACCN_F,), jnp.float32),
        pltpu.VMEM((_ACCC_F,), jnp.float32),
    ],
)
def _sc_partials(edge_hbm, node_hbm, batch_hbm, out_n, out_c, out_e,
                 ebuf, nbuf, bbuf, accn, accc):
    c = lax.axis_index("c")
    s = lax.axis_index("s")
    w = s * 2 + c

    zero16 = jnp.zeros((16,), jnp.float32)
    lane = lax.iota(jnp.int32, 16)
    ones16 = jnp.ones((16,), jnp.float32)

    ebase = w * (E_PER_W * EDGE_DIM)

    def eblk(g, acc):
        pltpu.sync_copy(
            edge_hbm.at[pl.ds(ebase + g * (EB * EDGE_DIM), EB * EDGE_DIM)],
            ebuf)

        def erow(i, acc):
            base = i * 128
            return tuple(acc[k] + ebuf[pl.ds(base + k * 16, 16)]
                         for k in range(8))

        return lax.fori_loop(0, EB // 8, erow, acc)

    accs = lax.fori_loop(0, E_BLOCKS, eblk, (zero16,) * 8)
    etot = accs[0]
    for k in range(1, 8):
        etot = etot + accs[k]
    ebuf[pl.ds(0, 16)] = etot
    for k in range(1, 8):
        ebuf[pl.ds(k * 16, 16)] = zero16
    pltpu.sync_copy(ebuf.at[pl.ds(0, 128)], out_e.at[w])

    pltpu.sync_copy(batch_hbm.at[w], bbuf)

    def zseg(j, _):
        accn[pl.ds(j * 16, 16)] = zero16
        return 0

    lax.fori_loop(0, _ACCN_F // 16, zseg, 0)

    def zcnt(j, _):
        accc[pl.ds(j * 16, 16)] = zero16
        return 0

    lax.fori_loop(0, _ACCC_F // 16, zcnt, 0)

    nbase = w * N_PER_W

    def nblk(g, _):
        nrows = jnp.where(g == N_BLOCKS - 1, N_LAST, NB)
        pltpu.sync_copy(
            node_hbm.at[pl.ds((nbase + g * NB) * NODE_DIM, nrows * NODE_DIM)],
            nbuf.at[pl.ds(0, nrows * NODE_DIM)])

        def ngrp(t, _):
            sids = bbuf[pl.ds(g * NB + t * 16, 16)]
            for l in range(16):
                sid = sids[l]
                idx0 = sid * NODE_DIM + lane
                rowbase = (t * 16 + l) * NODE_DIM
                for j in range(8):
                    plsc.addupdate_scatter(
                        accn, [idx0 + j * 16],
                        nbuf[pl.ds(rowbase + j * 16, 16)])
                plsc.addupdate_scatter(accc, [sid * 16 + lane], ones16)
            return 0

        return lax.fori_loop(0, NB // 16, ngrp, 0)

    lax.fori_loop(0, N_BLOCKS, nblk, 0)

    pltpu.sync_copy(accn.at[pl.ds(0, _SEG_F)], out_n.at[w])
    pltpu.sync_copy(accc.at[pl.ds(0, _CNT_F)], out_c.at[w])


def _tc_mlp(u_ref, np_ref, nc_ref, ep_ref, w1u_ref, w1n_ref, w1e_ref,
            b1_ref, w2_ref, b2_ref, o_ref):
    nagg = jnp.sum(np_ref[...], axis=0)
    cnt = jnp.sum(nc_ref[...], axis=0)[:, 0:1]
    cnt = jnp.maximum(cnt, 1.0)
    nagg = nagg / cnt
    esum = jnp.sum(ep_ref[...], axis=0, keepdims=True)[:, :16]
    emean = esum * (1.0 / N_EDGES)
    hp = jax.lax.Precision.HIGHEST
    h = (jnp.dot(u_ref[...], w1u_ref[...], precision=hp,
                 preferred_element_type=jnp.float32)
         + jnp.dot(nagg, w1n_ref[...], precision=hp,
                   preferred_element_type=jnp.float32)
         + jnp.dot(emean, w1e_ref[...], precision=hp,
                   preferred_element_type=jnp.float32)
         + b1_ref[...])
    h = jnp.maximum(h, 0.0)
    o_ref[...] = (jnp.dot(h, w2_ref[...], precision=hp,
                          preferred_element_type=jnp.float32)
                  + b2_ref[...])


def kernel(node_attr, edge_attr, u, batch, W1, b1, W2, b2):
    edge_flat = edge_attr.reshape(-1)
    node_flat = node_attr.reshape(-1)
    b2d = batch.reshape(NW, N_PER_W).astype(jnp.int32)
    b2d = jnp.pad(b2d, ((0, 0), (0, N_PAD_W - N_PER_W)),
                  constant_values=NUM_GRAPHS)

    out_n, out_c, out_e = _sc_partials(edge_flat, node_flat, b2d)

    np3 = out_n.reshape(NW, NUM_GRAPHS, NODE_DIM)
    nc3 = out_c.reshape(NW, NUM_GRAPHS, 16)

    w1u_t = W1[:, :NODE_DIM].T
    w1n_t = W1[:, NODE_DIM:2 * NODE_DIM].T
    w1e_t = W1[:, 2 * NODE_DIM:].T
    w2_t = W2.T
    b1r = b1.reshape(1, LATENT)
    b2r = b2.reshape(1, LATENT)

    return pl.pallas_call(
        _tc_mlp,
        out_shape=jax.ShapeDtypeStruct((NUM_GRAPHS, LATENT), jnp.float32),
    )(u, np3, nc3, out_e, w1u_t, w1n_t, w1e_t, b1r, w2_t, b2r)


# speedup vs baseline: 4.8954x; 4.8954x over previous
import functools

import jax
import jax.numpy as jnp
from jax import lax
from jax.experimental import pallas as pl
from jax.experimental.pallas import tpu as pltpu
from jax.experimental.pallas import tpu_sc as plsc

N_NODES = 100000
N_EDGES = 3200000
NODE_DIM = 128
EDGE_DIM = 16
NUM_GRAPHS = 64
LATENT = 128

NW = 32

E_PER_W = 781 * 128
E_LAST_W = N_EDGES - 31 * E_PER_W
EB = 4096
E_BLOCKS = -(-E_LAST_W // EB)

N_PER_W = 3128
N_LAST_W = N_NODES - 31 * N_PER_W
N_PAD_W = 3200
NB = 400
N_BLOCKS = N_PAD_W // NB

_SEG_F = NUM_GRAPHS * NODE_DIM
_CNT_F = NUM_GRAPHS * 16
_ACCN_F = (NUM_GRAPHS + 1) * NODE_DIM
_ACCC_F = (NUM_GRAPHS + 1) * 16


@functools.partial(
    pl.kernel,
    mesh=plsc.VectorSubcoreMesh(core_axis_name="c", subcore_axis_name="s"),
    compiler_params=pltpu.CompilerParams(needs_layout_passes=False),
    out_type=[
        jax.ShapeDtypeStruct((NW, _SEG_F), jnp.float32),
        jax.ShapeDtypeStruct((NW, _CNT_F), jnp.float32),
        jax.ShapeDtypeStruct((NW, EDGE_DIM * 16), jnp.float32),
    ],
    scratch_types=[
        pltpu.VMEM((EDGE_DIM, EB), jnp.float32),
        pltpu.VMEM((NB, NODE_DIM), jnp.float32),
        pltpu.VMEM((N_PAD_W,), jnp.int32),
        pltpu.VMEM((_ACCN_F,), jnp.float32),
        pltpu.VMEM((_ACCC_F,), jnp.float32),
        pltpu.VMEM((EDGE_DIM * 16,), jnp.float32),
    ],
)
def _sc_partials(edge_hbm, node_hbm, batch_hbm, out_n, out_c, out_e,
                 ebuf, nbuf, bbuf, accn, accc, estage):
    c = lax.axis_index("c")
    s = lax.axis_index("s")
    w = s * 2 + c

    zero16 = jnp.zeros((16,), jnp.float32)
    lane = lax.iota(jnp.int32, 16)
    ones16 = jnp.ones((16,), jnp.float32)

    ebase = w * E_PER_W
    ecols = jnp.where(w == NW - 1, E_LAST_W, E_PER_W)

    def eblk(g, acc):
        csize = pl.multiple_of(jnp.clip(ecols - g * EB, 0, EB), 128)

        @pl.when(csize > 0)
        def _():
            pltpu.sync_copy(
                edge_hbm.at[:, pl.ds(ebase + g * EB, csize)],
                ebuf.at[:, pl.ds(0, csize)])

        def ecol(j, acc):
            return tuple(acc[f] + ebuf[f, pl.ds(j * 16, 16)]
                         for f in range(EDGE_DIM))

        return lax.fori_loop(0, csize // 16, ecol, acc)

    accs = lax.fori_loop(0, E_BLOCKS, eblk, (zero16,) * EDGE_DIM)
    for f in range(EDGE_DIM):
        estage[pl.ds(f * 16, 16)] = accs[f]
    pltpu.sync_copy(estage, out_e.at[w])

    pltpu.sync_copy(batch_hbm.at[w], bbuf)

    def zseg(j, _):
        accn[pl.ds(j * 16, 16)] = zero16
        return 0

    lax.fori_loop(0, _ACCN_F // 16, zseg, 0)

    def zcnt(j, _):
        accc[pl.ds(j * 16, 16)] = zero16
        return 0

    lax.fori_loop(0, _CNT_F // 16, zcnt, 0)

    nbase = w * N_PER_W
    rows_w = jnp.where(w == NW - 1, N_LAST_W, N_PER_W)

    def nblk(g, _):
        nrows = pl.multiple_of(jnp.clip(rows_w - g * NB, 0, NB), 8)

        @pl.when(nrows > 0)
        def _():
            pltpu.sync_copy(
                node_hbm.at[pl.ds(nbase + g * NB, nrows)],
                nbuf.at[pl.ds(0, nrows)])

        def ngrp(t, _):
            sids = bbuf[pl.ds(g * NB + t * 16, 16)]
            for l in range(16):
                sid = sids[l]
                idx0 = sid * NODE_DIM + lane
                row = t * 16 + l
                for j in range(8):
                    plsc.addupdate_scatter(
                        accn, [idx0 + j * 16],
                        nbuf[row, pl.ds(j * 16, 16)])
                plsc.addupdate_scatter(accc, [sid * 16 + lane], ones16)
            return 0

        return lax.fori_loop(0, NB // 16, ngrp, 0)

    lax.fori_loop(0, N_BLOCKS, nblk, 0)

    pltpu.sync_copy(accn.at[pl.ds(0, _SEG_F)], out_n.at[w])
    pltpu.sync_copy(accc.at[pl.ds(0, _CNT_F)], out_c.at[w])


def _tc_mlp(u_ref, np_ref, nc_ref, ep_ref, w1u_ref, w1n_ref, w1e_ref,
            b1_ref, w2_ref, b2_ref, o_ref):
    nagg = jnp.sum(np_ref[...], axis=0)
    cnt = jnp.sum(nc_ref[...], axis=0)[:, 0:1]
    cnt = jnp.maximum(cnt, 1.0)
    nagg = nagg / cnt
    esum = jnp.sum(ep_ref[...], axis=(0, 2))[None, :]
    emean = esum * (1.0 / N_EDGES)
    hp = jax.lax.Precision.HIGHEST
    h = (jnp.dot(u_ref[...], w1u_ref[...], precision=hp,
                 preferred_element_type=jnp.float32)
         + jnp.dot(nagg, w1n_ref[...], precision=hp,
                   preferred_element_type=jnp.float32)
         + jnp.dot(emean, w1e_ref[...], precision=hp,
                   preferred_element_type=jnp.float32)
         + b1_ref[...])
    h = jnp.maximum(h, 0.0)
    o_ref[...] = (jnp.dot(h, w2_ref[...], precision=hp,
                          preferred_element_type=jnp.float32)
                  + b2_ref[...])


def kernel(node_attr, edge_attr, u, batch, W1, b1, W2, b2):
    bpad = jnp.concatenate(
        [batch.astype(jnp.int32),
         jnp.full((NW * N_PAD_W - N_NODES,), NUM_GRAPHS, jnp.int32)])
    widx = jnp.arange(NW, dtype=jnp.int32)[:, None]
    kidx = jnp.arange(N_PAD_W, dtype=jnp.int32)[None, :]
    rows_per_w = jnp.where(widx == NW - 1, N_LAST_W, N_PER_W)
    b2d = jnp.where(kidx < rows_per_w,
                    bpad[widx * N_PER_W + kidx],
                    jnp.int32(NUM_GRAPHS))

    out_n, out_c, out_e = _sc_partials(edge_attr.T, node_attr, b2d)

    np3 = out_n.reshape(NW, NUM_GRAPHS, NODE_DIM)
    nc3 = out_c.reshape(NW, NUM_GRAPHS, 16)
    ep3 = out_e.reshape(NW, EDGE_DIM, 16)

    w1u_t = W1[:, :NODE_DIM].T
    w1n_t = W1[:, NODE_DIM:2 * NODE_DIM].T
    w1e_t = W1[:, 2 * NODE_DIM:].T
    w2_t = W2.T
    b1r = b1.reshape(1, LATENT)
    b2r = b2.reshape(1, LATENT)

    return pl.pallas_call(
        _tc_mlp,
        out_shape=jax.ShapeDtypeStruct((NUM_GRAPHS, LATENT), jnp.float32),
    )(u, np3, nc3, ep3, w1u_t, w1n_t, w1e_t, b1r, w2_t, b2r)


# speedup vs baseline: 6.5612x; 1.3403x over previous
import functools

import jax
import jax.numpy as jnp
from jax import lax
from jax.experimental import pallas as pl
from jax.experimental.pallas import tpu as pltpu
from jax.experimental.pallas import tpu_sc as plsc

N_NODES = 100000
N_EDGES = 3200000
NODE_DIM = 128
EDGE_DIM = 16
NUM_GRAPHS = 64
LATENT = 128

NW = 32

E_PER_W = 781 * 128
E_LAST_W = N_EDGES - 31 * E_PER_W
EB = 2048
E_BLOCKS = 50

N_PER_W = 3128
N_LAST_W = N_NODES - 31 * N_PER_W
N_PAD_W = 3200
NB = 160
N_BLOCKS = N_PAD_W // NB

_SEG_F = NUM_GRAPHS * NODE_DIM
_CNT_F = NUM_GRAPHS * 16
_ACCN_F = (NUM_GRAPHS + 1) * NODE_DIM
_ACCC_F = (NUM_GRAPHS + 1) * 16


@functools.partial(
    pl.kernel,
    mesh=plsc.VectorSubcoreMesh(core_axis_name="c", subcore_axis_name="s"),
    compiler_params=pltpu.CompilerParams(needs_layout_passes=False),
    out_type=[
        jax.ShapeDtypeStruct((NW, _SEG_F), jnp.float32),
        jax.ShapeDtypeStruct((NW, _CNT_F), jnp.float32),
        jax.ShapeDtypeStruct((NW, EDGE_DIM * 16), jnp.float32),
    ],
    scratch_types=[
        pltpu.VMEM((EDGE_DIM, EB), jnp.float32),
        pltpu.VMEM((EDGE_DIM, EB), jnp.float32),
        pltpu.VMEM((NB, NODE_DIM), jnp.float32),
        pltpu.VMEM((NB, NODE_DIM), jnp.float32),
        pltpu.VMEM((N_PAD_W,), jnp.int32),
        pltpu.VMEM((_ACCN_F,), jnp.float32),
        pltpu.VMEM((_ACCC_F,), jnp.float32),
        pltpu.VMEM((EDGE_DIM * 16,), jnp.float32),
        pltpu.SemaphoreType.DMA,
        pltpu.SemaphoreType.DMA,
        pltpu.SemaphoreType.DMA,
    ],
)
def _sc_partials(edge_hbm, node_hbm, batch_hbm, out_n, out_c, out_e,
                 ebuf0, ebuf1, nbuf0, nbuf1, bbuf, accn, accc, estage,
                 sem0, sem1, semb):
    c = lax.axis_index("c")
    s = lax.axis_index("s")
    w = s * 2 + c

    zero16 = jnp.zeros((16,), jnp.float32)
    lane = lax.iota(jnp.int32, 16)
    ones16 = jnp.ones((16,), jnp.float32)

    pltpu.async_copy(batch_hbm.at[w], bbuf, semb)

    ebase = w * E_PER_W
    ecols = jnp.where(w == NW - 1, E_LAST_W, E_PER_W)

    def ecsize(g):
        return pl.multiple_of(jnp.clip(ecols - g * EB, 0, EB), 128)

    def e_pair(g, buf, sem):
        csize = ecsize(g)
        return (edge_hbm.at[:, pl.ds(ebase + g * EB, csize)],
                buf.at[:, pl.ds(0, csize)], csize)

    def e_start(g, buf, sem):
        src, dst, csize = e_pair(g, buf, sem)

        @pl.when(csize > 0)
        def _():
            pltpu.async_copy(src, dst, sem)

    def e_wait(g, buf, sem):
        src, dst, csize = e_pair(g, buf, sem)

        @pl.when(csize > 0)
        def _():
            pltpu.make_async_copy(src, dst, sem).wait()

    def e_compute(g, buf, acc):
        csize = ecsize(g)

        def ecol(j, acc):
            return tuple(acc[f] + buf[f, pl.ds(j * 16, 16)]
                         for f in range(EDGE_DIM))

        return lax.fori_loop(0, csize // 16, ecol, acc)

    e_start(0, ebuf0, sem0)

    def zseg(j, _):
        accn[pl.ds(j * 16, 16)] = zero16
        return 0

    lax.fori_loop(0, _ACCN_F // 16, zseg, 0)

    def zcnt(j, _):
        accc[pl.ds(j * 16, 16)] = zero16
        return 0

    lax.fori_loop(0, _CNT_F // 16, zcnt, 0)

    def epair_body(p, acc):
        g0 = 2 * p
        e_start(g0 + 1, ebuf1, sem1)
        e_wait(g0, ebuf0, sem0)
        acc = e_compute(g0, ebuf0, acc)
        e_start(g0 + 2, ebuf0, sem0)
        e_wait(g0 + 1, ebuf1, sem1)
        return e_compute(g0 + 1, ebuf1, acc)

    accs = lax.fori_loop(0, E_BLOCKS // 2, epair_body, (zero16,) * EDGE_DIM)
    for f in range(EDGE_DIM):
        estage[pl.ds(f * 16, 16)] = accs[f]
    pltpu.sync_copy(estage, out_e.at[w])

    nbase = w * N_PER_W
    rows_w = jnp.where(w == NW - 1, N_LAST_W, N_PER_W)

    def nsize(g):
        return pl.multiple_of(jnp.clip(rows_w - g * NB, 0, NB), 8)

    def n_pair(g, buf):
        nrows = nsize(g)
        return (node_hbm.at[pl.ds(nbase + g * NB, nrows)],
                buf.at[pl.ds(0, nrows)], nrows)

    def n_start(g, buf, sem):
        src, dst, nrows = n_pair(g, buf)

        @pl.when(nrows > 0)
        def _():
            pltpu.async_copy(src, dst, sem)

    def n_wait(g, buf, sem):
        src, dst, nrows = n_pair(g, buf)

        @pl.when(nrows > 0)
        def _():
            pltpu.make_async_copy(src, dst, sem).wait()

    def n_compute(g, buf):
        def ngrp(t, _):
            sids = bbuf[pl.ds(g * NB + t * 16, 16)]
            for l in range(16):
                sid = sids[l]
                idx0 = sid * NODE_DIM + lane
                row = t * 16 + l
                for j in range(8):
                    plsc.addupdate_scatter(
                        accn, [idx0 + j * 16],
                        buf[row, pl.ds(j * 16, 16)])
                plsc.addupdate_scatter(accc, [sid * 16 + lane], ones16)
            return 0

        lax.fori_loop(0, NB // 16, ngrp, 0)

    n_start(0, nbuf0, sem0)
    pltpu.make_async_copy(batch_hbm.at[w], bbuf, semb).wait()

    def npair_body(p, _):
        g0 = 2 * p
        n_start(g0 + 1, nbuf1, sem1)
        n_wait(g0, nbuf0, sem0)
        n_compute(g0, nbuf0)
        n_start(g0 + 2, nbuf0, sem0)
        n_wait(g0 + 1, nbuf1, sem1)
        n_compute(g0 + 1, nbuf1)
        return 0

    lax.fori_loop(0, N_BLOCKS // 2, npair_body, 0)

    pltpu.sync_copy(accn.at[pl.ds(0, _SEG_F)], out_n.at[w])
    pltpu.sync_copy(accc.at[pl.ds(0, _CNT_F)], out_c.at[w])


def _tc_mlp(u_ref, np_ref, nc_ref, ep_ref, w1u_ref, w1n_ref, w1e_ref,
            b1_ref, w2_ref, b2_ref, o_ref):
    nagg = jnp.sum(np_ref[...], axis=0)
    cnt = jnp.sum(nc_ref[...], axis=0)[:, 0:1]
    cnt = jnp.maximum(cnt, 1.0)
    nagg = nagg / cnt
    esum = jnp.sum(ep_ref[...], axis=(0, 2))[None, :]
    emean = esum * (1.0 / N_EDGES)
    hp = jax.lax.Precision.HIGHEST
    h = (jnp.dot(u_ref[...], w1u_ref[...], precision=hp,
                 preferred_element_type=jnp.float32)
         + jnp.dot(nagg, w1n_ref[...], precision=hp,
                   preferred_element_type=jnp.float32)
         + jnp.dot(emean, w1e_ref[...], precision=hp,
                   preferred_element_type=jnp.float32)
         + b1_ref[...])
    h = jnp.maximum(h, 0.0)
    o_ref[...] = (jnp.dot(h, w2_ref[...], precision=hp,
                          preferred_element_type=jnp.float32)
                  + b2_ref[...])


def kernel(node_attr, edge_attr, u, batch, W1, b1, W2, b2):
    bpad = jnp.concatenate(
        [batch.astype(jnp.int32),
         jnp.full((NW * N_PAD_W - N_NODES,), NUM_GRAPHS, jnp.int32)])
    widx = jnp.arange(NW, dtype=jnp.int32)[:, None]
    kidx = jnp.arange(N_PAD_W, dtype=jnp.int32)[None, :]
    rows_per_w = jnp.where(widx == NW - 1, N_LAST_W, N_PER_W)
    b2d = jnp.where(kidx < rows_per_w,
                    bpad[widx * N_PER_W + kidx],
                    jnp.int32(NUM_GRAPHS))

    out_n, out_c, out_e = _sc_partials(edge_attr.T, node_attr, b2d)

    np3 = out_n.reshape(NW, NUM_GRAPHS, NODE_DIM)
    nc3 = out_c.reshape(NW, NUM_GRAPHS, 16)
    ep3 = out_e.reshape(NW, EDGE_DIM, 16)

    w1u_t = W1[:, :NODE_DIM].T
    w1n_t = W1[:, NODE_DIM:2 * NODE_DIM].T
    w1e_t = W1[:, 2 * NODE_DIM:].T
    w2_t = W2.T
    b1r = b1.reshape(1, LATENT)
    b2r = b2.reshape(1, LATENT)

    return pl.pallas_call(
        _tc_mlp,
        out_shape=jax.ShapeDtypeStruct((NUM_GRAPHS, LATENT), jnp.float32),
    )(u, np3, nc3, ep3, w1u_t, w1n_t, w1e_t, b1r, w2_t, b2r)


# speedup vs baseline: 10.5023x; 1.6007x over previous
import functools

import jax
import jax.numpy as jnp
from jax import lax
from jax.experimental import pallas as pl
from jax.experimental.pallas import tpu as pltpu
from jax.experimental.pallas import tpu_sc as plsc

N_NODES = 100000
N_EDGES = 3200000
NODE_DIM = 128
EDGE_DIM = 16
NUM_GRAPHS = 64
LATENT = 128

NW = 32

N_PER_W = 3128
N_LAST_W = N_NODES - 31 * N_PER_W
N_PAD_W = 3200
NB = 160
N_BLOCKS = N_PAD_W // NB

EB_TC = 32000

_SEG_F = NUM_GRAPHS * NODE_DIM
_CNT_F = NUM_GRAPHS * 16
_ACCN_F = (NUM_GRAPHS + 1) * NODE_DIM
_ACCC_F = (NUM_GRAPHS + 1) * 16


@functools.partial(
    pl.kernel,
    mesh=plsc.VectorSubcoreMesh(core_axis_name="c", subcore_axis_name="s"),
    compiler_params=pltpu.CompilerParams(needs_layout_passes=False),
    out_type=[
        jax.ShapeDtypeStruct((NW, _SEG_F), jnp.float32),
        jax.ShapeDtypeStruct((NW, _CNT_F), jnp.float32),
    ],
    scratch_types=[
        pltpu.VMEM((NB, NODE_DIM), jnp.float32),
        pltpu.VMEM((NB, NODE_DIM), jnp.float32),
        pltpu.VMEM((N_PAD_W,), jnp.int32),
        pltpu.VMEM((_ACCN_F,), jnp.float32),
        pltpu.VMEM((_ACCC_F,), jnp.float32),
        pltpu.SemaphoreType.DMA,
        pltpu.SemaphoreType.DMA,
        pltpu.SemaphoreType.DMA,
    ],
)
def _sc_segsum(node_hbm, batch_hbm, out_n, out_c,
               nbuf0, nbuf1, bbuf, accn, accc, sem0, sem1, semb):
    c = lax.axis_index("c")
    s = lax.axis_index("s")
    w = s * 2 + c

    zero16 = jnp.zeros((16,), jnp.float32)
    lane = lax.iota(jnp.int32, 16)
    ones16 = jnp.ones((16,), jnp.float32)
    pad16 = jnp.full((16,), NUM_GRAPHS, jnp.int32)

    nbase = w * N_PER_W
    rows_w = jnp.where(w == NW - 1, N_LAST_W, N_PER_W)
    rows_w = pl.multiple_of(rows_w, 8)

    for t in range(3024 // 16, N_PAD_W // 16):
        bbuf[pl.ds(t * 16, 16)] = pad16
    pltpu.async_copy(batch_hbm.at[pl.ds(nbase, rows_w)],
                     bbuf.at[pl.ds(0, rows_w)], semb)

    def nsize(g):
        return pl.multiple_of(jnp.clip(rows_w - g * NB, 0, NB), 8)

    def n_pair(g, buf):
        nrows = nsize(g)
        return (node_hbm.at[pl.ds(nbase + g * NB, nrows)],
                buf.at[pl.ds(0, nrows)], nrows)

    def n_start(g, buf, sem):
        src, dst, nrows = n_pair(g, buf)

        @pl.when(nrows > 0)
        def _():
            pltpu.async_copy(src, dst, sem)

    def n_wait(g, buf, sem):
        src, dst, nrows = n_pair(g, buf)

        @pl.when(nrows > 0)
        def _():
            pltpu.make_async_copy(src, dst, sem).wait()

    def n_compute(g, buf):
        def ngrp(t, _):
            sids = bbuf[pl.ds(g * NB + t * 16, 16)]
            for l in range(16):
                sid = sids[l]
                idx0 = sid * NODE_DIM + lane
                row = t * 16 + l
                for j in range(8):
                    plsc.addupdate_scatter(
                        accn, [idx0 + j * 16],
                        buf[row, pl.ds(j * 16, 16)])
                plsc.addupdate_scatter(accc, [sid * 16 + lane], ones16)
            return 0

        lax.fori_loop(0, NB // 16, ngrp, 0)

    n_start(0, nbuf0, sem0)

    def zseg(j, _):
        accn[pl.ds(j * 16, 16)] = zero16
        return 0

    lax.fori_loop(0, _ACCN_F // 16, zseg, 0)

    def zcnt(j, _):
        accc[pl.ds(j * 16, 16)] = zero16
        return 0

    lax.fori_loop(0, _ACCC_F // 16, zcnt, 0)

    pltpu.make_async_copy(batch_hbm.at[pl.ds(nbase, rows_w)],
                          bbuf.at[pl.ds(0, rows_w)], semb).wait()

    def npair_body(p, _):
        g0 = 2 * p
        n_start(g0 + 1, nbuf1, sem1)
        n_wait(g0, nbuf0, sem0)
        n_compute(g0, nbuf0)
        n_start(g0 + 2, nbuf0, sem0)
        n_wait(g0 + 1, nbuf1, sem1)
        n_compute(g0 + 1, nbuf1)
        return 0

    lax.fori_loop(0, N_BLOCKS // 2, npair_body, 0)

    pltpu.sync_copy(accn.at[pl.ds(0, _SEG_F)], out_n.at[w])
    pltpu.sync_copy(accc.at[pl.ds(0, _CNT_F)], out_c.at[w])


def _tc_edge_reduce(e_ref, o_ref):
    i = pl.program_id(0)
    x = e_ref[...]
    partial = jnp.sum(x.reshape(EDGE_DIM, EB_TC // 128, 128), axis=1)

    @pl.when(i == 0)
    def _():
        o_ref[...] = partial

    @pl.when(i > 0)
    def _():
        o_ref[...] += partial


def _tc_mlp(u_ref, np_ref, nc_ref, ep_ref, w1u_ref, w1n_ref, w1e_ref,
            b1_ref, w2_ref, b2_ref, o_ref):
    nagg = jnp.sum(np_ref[...], axis=0)
    cnt = jnp.sum(nc_ref[...], axis=0)[:, 0:1]
    cnt = jnp.maximum(cnt, 1.0)
    nagg = nagg / cnt
    esum = jnp.sum(ep_ref[...], axis=1)[None, :]
    emean = esum * (1.0 / N_EDGES)
    hp = jax.lax.Precision.HIGHEST
    h = (jnp.dot(u_ref[...], w1u_ref[...], precision=hp,
                 preferred_element_type=jnp.float32)
         + jnp.dot(nagg, w1n_ref[...], precision=hp,
                   preferred_element_type=jnp.float32)
         + jnp.dot(emean, w1e_ref[...], precision=hp,
                   preferred_element_type=jnp.float32)
         + b1_ref[...])
    h = jnp.maximum(h, 0.0)
    o_ref[...] = (jnp.dot(h, w2_ref[...], precision=hp,
                          preferred_element_type=jnp.float32)
                  + b2_ref[...])


def kernel(node_attr, edge_attr, u, batch, W1, b1, W2, b2):
    batch32 = batch.astype(jnp.int32)

    out_n, out_c = _sc_segsum(node_attr, batch32)

    edge_t = edge_attr.T
    ep = pl.pallas_call(
        _tc_edge_reduce,
        grid=(N_EDGES // EB_TC,),
        in_specs=[pl.BlockSpec((EDGE_DIM, EB_TC), lambda i: (0, i))],
        out_specs=pl.BlockSpec((EDGE_DIM, 128), lambda i: (0, 0)),
        out_shape=jax.ShapeDtypeStruct((EDGE_DIM, 128), jnp.float32),
    )(edge_t)

    np3 = out_n.reshape(NW, NUM_GRAPHS, NODE_DIM)
    nc3 = out_c.reshape(NW, NUM_GRAPHS, 16)

    w1u_t = W1[:, :NODE_DIM].T
    w1n_t = W1[:, NODE_DIM:2 * NODE_DIM].T
    w1e_t = W1[:, 2 * NODE_DIM:].T
    w2_t = W2.T
    b1r = b1.reshape(1, LATENT)
    b2r = b2.reshape(1, LATENT)

    return pl.pallas_call(
        _tc_mlp,
        out_shape=jax.ShapeDtypeStruct((NUM_GRAPHS, LATENT), jnp.float32),
    )(u, np3, nc3, ep, w1u_t, w1n_t, w1e_t, b1r, w2_t, b2r)


# speedup vs baseline: 10.7345x; 1.0221x over previous
import functools

import jax
import jax.numpy as jnp
from jax import lax
from jax.experimental import pallas as pl
from jax.experimental.pallas import tpu as pltpu
from jax.experimental.pallas import tpu_sc as plsc

N_NODES = 100000
N_EDGES = 3200000
NODE_DIM = 128
EDGE_DIM = 16
NUM_GRAPHS = 64
LATENT = 128

NW = 32

N_PER_W = 3128
N_LAST_W = N_NODES - 31 * N_PER_W
N_PAD_W = 3200
NB = 160
N_BLOCKS = N_PAD_W // NB

EB_TC = 32000

_SEG_F = NUM_GRAPHS * NODE_DIM
_CNT_F = NUM_GRAPHS * 16
_ACCN_F = (NUM_GRAPHS + 1) * NODE_DIM
_ACCC_F = (NUM_GRAPHS + 1) * 16


@functools.partial(
    pl.kernel,
    mesh=plsc.VectorSubcoreMesh(core_axis_name="c", subcore_axis_name="s"),
    compiler_params=pltpu.CompilerParams(needs_layout_passes=False),
    out_type=[
        jax.ShapeDtypeStruct((NW, _SEG_F), jnp.float32),
        jax.ShapeDtypeStruct((NW, _CNT_F), jnp.float32),
    ],
    scratch_types=[
        pltpu.VMEM((NB, NODE_DIM), jnp.float32),
        pltpu.VMEM((NB, NODE_DIM), jnp.float32),
        pltpu.VMEM((N_PAD_W,), jnp.int32),
        pltpu.VMEM((_ACCN_F,), jnp.float32),
        pltpu.VMEM((_ACCC_F,), jnp.float32),
        pltpu.SemaphoreType.DMA,
        pltpu.SemaphoreType.DMA,
        pltpu.SemaphoreType.DMA,
    ],
)
def _sc_segsum(node_hbm, batch_hbm, out_n, out_c,
               nbuf0, nbuf1, bbuf, accn, accc, sem0, sem1, semb):
    c = lax.axis_index("c")
    s = lax.axis_index("s")
    w = s * 2 + c

    zero16 = jnp.zeros((16,), jnp.float32)
    lane = lax.iota(jnp.int32, 16)
    ones16 = jnp.ones((16,), jnp.float32)
    pad16 = jnp.full((16,), NUM_GRAPHS, jnp.int32)

    nbase = w * N_PER_W
    rows_w = jnp.where(w == NW - 1, N_LAST_W, N_PER_W)
    rows_w = pl.multiple_of(rows_w, 8)

    for t in range(3024 // 16, N_PAD_W // 16):
        bbuf[pl.ds(t * 16, 16)] = pad16
    pltpu.async_copy(batch_hbm.at[pl.ds(nbase, rows_w)],
                     bbuf.at[pl.ds(0, rows_w)], semb)

    def nsize(g):
        return pl.multiple_of(jnp.clip(rows_w - g * NB, 0, NB), 8)

    def n_pair(g, buf):
        nrows = nsize(g)
        return (node_hbm.at[pl.ds(nbase + g * NB, nrows)],
                buf.at[pl.ds(0, nrows)], nrows)

    def n_start(g, buf, sem):
        src, dst, nrows = n_pair(g, buf)

        @pl.when(nrows > 0)
        def _():
            pltpu.async_copy(src, dst, sem)

    def n_wait(g, buf, sem):
        src, dst, nrows = n_pair(g, buf)

        @pl.when(nrows > 0)
        def _():
            pltpu.make_async_copy(src, dst, sem).wait()

    def flush_run(prev_sid, cnt, accs):
        idx0 = prev_sid * NODE_DIM + lane

        @pl.when(prev_sid < NUM_GRAPHS)
        def _():
            for j in range(8):
                plsc.addupdate_scatter(accn, [idx0 + j * 16], accs[j])
            plsc.addupdate_scatter(accc, [prev_sid * 16 + lane], cnt)

    def n_compute(g, buf, carry):
        def ngrp(t, carry):
            prev_sid, cnt, accs = carry
            sids = bbuf[pl.ds(g * NB + t * 16, 16)]
            for l in range(16):
                sid = sids[l]
                flush = sid != prev_sid
                row = t * 16 + l
                rowv = tuple(buf[row, pl.ds(j * 16, 16)] for j in range(8))

                @pl.when(flush)
                def _(prev_sid=prev_sid, cnt=cnt, accs=accs):
                    flush_run(prev_sid, cnt, accs)

                accs = tuple(jnp.where(flush, rowv[j], accs[j] + rowv[j])
                             for j in range(8))
                cnt = jnp.where(flush, ones16, cnt + ones16)
                prev_sid = sid
            return (prev_sid, cnt, accs)

        return lax.fori_loop(0, NB // 16, ngrp, carry)

    n_start(0, nbuf0, sem0)

    def zseg(j, _):
        accn[pl.ds(j * 16, 16)] = zero16
        return 0

    lax.fori_loop(0, _ACCN_F // 16, zseg, 0)

    def zcnt(j, _):
        accc[pl.ds(j * 16, 16)] = zero16
        return 0

    lax.fori_loop(0, _ACCC_F // 16, zcnt, 0)

    pltpu.make_async_copy(batch_hbm.at[pl.ds(nbase, rows_w)],
                          bbuf.at[pl.ds(0, rows_w)], semb).wait()

    def npair_body(p, carry):
        g0 = 2 * p
        n_start(g0 + 1, nbuf1, sem1)
        n_wait(g0, nbuf0, sem0)
        carry = n_compute(g0, nbuf0, carry)
        n_start(g0 + 2, nbuf0, sem0)
        n_wait(g0 + 1, nbuf1, sem1)
        return n_compute(g0 + 1, nbuf1, carry)

    carry0 = (jnp.int32(NUM_GRAPHS), zero16, (zero16,) * 8)
    lax.fori_loop(0, N_BLOCKS // 2, npair_body, carry0)

    pltpu.sync_copy(accn.at[pl.ds(0, _SEG_F)], out_n.at[w])
    pltpu.sync_copy(accc.at[pl.ds(0, _CNT_F)], out_c.at[w])


def _tc_edge_reduce(e_ref, o_ref):
    i = pl.program_id(0)
    x = e_ref[...]
    partial = jnp.sum(x.reshape(EDGE_DIM, EB_TC // 128, 128), axis=1)

    @pl.when(i == 0)
    def _():
        o_ref[...] = partial

    @pl.when(i > 0)
    def _():
        o_ref[...] += partial


def _tc_mlp(u_ref, np_ref, nc_ref, ep_ref, w1u_ref, w1n_ref, w1e_ref,
            b1_ref, w2_ref, b2_ref, o_ref):
    nagg = jnp.sum(np_ref[...], axis=0)
    cnt = jnp.sum(nc_ref[...], axis=0)[:, 0:1]
    cnt = jnp.maximum(cnt, 1.0)
    nagg = nagg / cnt
    esum = jnp.sum(ep_ref[...], axis=1)[None, :]
    emean = esum * (1.0 / N_EDGES)
    hp = jax.lax.Precision.HIGHEST
    h = (jnp.dot(u_ref[...], w1u_ref[...], precision=hp,
                 preferred_element_type=jnp.float32)
         + jnp.dot(nagg, w1n_ref[...], precision=hp,
                   preferred_element_type=jnp.float32)
         + jnp.dot(emean, w1e_ref[...], precision=hp,
                   preferred_element_type=jnp.float32)
         + b1_ref[...])
    h = jnp.maximum(h, 0.0)
    o_ref[...] = (jnp.dot(h, w2_ref[...], precision=hp,
                          preferred_element_type=jnp.float32)
                  + b2_ref[...])


def kernel(node_attr, edge_attr, u, batch, W1, b1, W2, b2):
    batch32 = batch.astype(jnp.int32)

    out_n, out_c = _sc_segsum(node_attr, batch32)

    edge_t = edge_attr.T
    ep = pl.pallas_call(
        _tc_edge_reduce,
        grid=(N_EDGES // EB_TC,),
        in_specs=[pl.BlockSpec((EDGE_DIM, EB_TC), lambda i: (0, i))],
        out_specs=pl.BlockSpec((EDGE_DIM, 128), lambda i: (0, 0)),
        out_shape=jax.ShapeDtypeStruct((EDGE_DIM, 128), jnp.float32),
    )(edge_t)

    np3 = out_n.reshape(NW, NUM_GRAPHS, NODE_DIM)
    nc3 = out_c.reshape(NW, NUM_GRAPHS, 16)

    w1u_t = W1[:, :NODE_DIM].T
    w1n_t = W1[:, NODE_DIM:2 * NODE_DIM].T
    w1e_t = W1[:, 2 * NODE_DIM:].T
    w2_t = W2.T
    b1r = b1.reshape(1, LATENT)
    b2r = b2.reshape(1, LATENT)

    return pl.pallas_call(
        _tc_mlp,
        out_shape=jax.ShapeDtypeStruct((NUM_GRAPHS, LATENT), jnp.float32),
    )(u, np3, nc3, ep, w1u_t, w1n_t, w1e_t, b1r, w2_t, b2r)


# speedup vs baseline: 13.7878x; 1.2844x over previous
import functools

import jax
import jax.numpy as jnp
from jax import lax
from jax.experimental import pallas as pl
from jax.experimental.pallas import tpu as pltpu
from jax.experimental.pallas import tpu_sc as plsc

N_NODES = 100000
N_EDGES = 3200000
NODE_DIM = 128
EDGE_DIM = 16
NUM_GRAPHS = 64
LATENT = 128

NW = 32

N_PER_W = 3128
N_LAST_W = N_NODES - 31 * N_PER_W
N_PAD_W = 3200
NB = 160
N_BLOCKS = N_PAD_W // NB

EB_TC = 128000

_SEG_F = NUM_GRAPHS * NODE_DIM
_CNT_F = NUM_GRAPHS * 16
_ACCN_F = (NUM_GRAPHS + 1) * NODE_DIM
_ACCC_F = (NUM_GRAPHS + 1) * 16


@functools.partial(
    pl.kernel,
    mesh=plsc.VectorSubcoreMesh(core_axis_name="c", subcore_axis_name="s"),
    compiler_params=pltpu.CompilerParams(needs_layout_passes=False),
    out_type=[
        jax.ShapeDtypeStruct((NW, _SEG_F), jnp.float32),
        jax.ShapeDtypeStruct((NW, _CNT_F), jnp.float32),
    ],
    scratch_types=[
        pltpu.VMEM((NB, NODE_DIM), jnp.float32),
        pltpu.VMEM((NB, NODE_DIM), jnp.float32),
        pltpu.VMEM((N_PAD_W,), jnp.int32),
        pltpu.VMEM((_ACCN_F,), jnp.float32),
        pltpu.VMEM((_ACCC_F,), jnp.float32),
        pltpu.SemaphoreType.DMA,
        pltpu.SemaphoreType.DMA,
        pltpu.SemaphoreType.DMA,
    ],
)
def _sc_segsum(node_hbm, batch_hbm, out_n, out_c,
               nbuf0, nbuf1, bbuf, accn, accc, sem0, sem1, semb):
    c = lax.axis_index("c")
    s = lax.axis_index("s")
    w = s * 2 + c

    zero16 = jnp.zeros((16,), jnp.float32)
    lane = lax.iota(jnp.int32, 16)
    ones16 = jnp.ones((16,), jnp.float32)
    pad16 = jnp.full((16,), NUM_GRAPHS, jnp.int32)

    nbase = w * N_PER_W
    rows_w = jnp.where(w == NW - 1, N_LAST_W, N_PER_W)
    rows_w = pl.multiple_of(rows_w, 8)

    for t in range(3024 // 16, N_PAD_W // 16):
        bbuf[pl.ds(t * 16, 16)] = pad16
    pltpu.async_copy(batch_hbm.at[pl.ds(nbase, rows_w)],
                     bbuf.at[pl.ds(0, rows_w)], semb)

    def nsize(g):
        return pl.multiple_of(jnp.clip(rows_w - g * NB, 0, NB), 8)

    def n_pair(g, buf):
        nrows = nsize(g)
        return (node_hbm.at[pl.ds(nbase + g * NB, nrows)],
                buf.at[pl.ds(0, nrows)], nrows)

    def n_start(g, buf, sem):
        src, dst, nrows = n_pair(g, buf)

        @pl.when(nrows > 0)
        def _():
            pltpu.async_copy(src, dst, sem)

    def n_wait(g, buf, sem):
        src, dst, nrows = n_pair(g, buf)

        @pl.when(nrows > 0)
        def _():
            pltpu.make_async_copy(src, dst, sem).wait()

    def flush_run(prev_sid, cnt, accs):
        idx0 = prev_sid * NODE_DIM + lane

        @pl.when(prev_sid < NUM_GRAPHS)
        def _():
            for j in range(8):
                plsc.addupdate_scatter(accn, [idx0 + j * 16], accs[j])
            plsc.addupdate_scatter(accc, [prev_sid * 16 + lane], cnt)

    def n_compute(g, buf, carry):
        def ngrp(t, carry):
            prev_sid, cnt, accs = carry
            sids = bbuf[pl.ds(g * NB + t * 16, 16)]
            for l in range(16):
                sid = sids[l]
                flush = sid != prev_sid
                row = t * 16 + l
                rowv = tuple(buf[row, pl.ds(j * 16, 16)] for j in range(8))

                @pl.when(flush)
                def _(prev_sid=prev_sid, cnt=cnt, accs=accs):
                    flush_run(prev_sid, cnt, accs)

                accs = tuple(jnp.where(flush, rowv[j], accs[j] + rowv[j])
                             for j in range(8))
                cnt = jnp.where(flush, ones16, cnt + ones16)
                prev_sid = sid
            return (prev_sid, cnt, accs)

        return lax.fori_loop(0, NB // 16, ngrp, carry)

    n_start(0, nbuf0, sem0)

    def zseg(j, _):
        accn[pl.ds(j * 16, 16)] = zero16
        return 0

    lax.fori_loop(0, _ACCN_F // 16, zseg, 0)

    def zcnt(j, _):
        accc[pl.ds(j * 16, 16)] = zero16
        return 0

    lax.fori_loop(0, _ACCC_F // 16, zcnt, 0)

    pltpu.make_async_copy(batch_hbm.at[pl.ds(nbase, rows_w)],
                          bbuf.at[pl.ds(0, rows_w)], semb).wait()

    def npair_body(p, carry):
        g0 = 2 * p
        n_start(g0 + 1, nbuf1, sem1)
        n_wait(g0, nbuf0, sem0)
        carry = n_compute(g0, nbuf0, carry)
        n_start(g0 + 2, nbuf0, sem0)
        n_wait(g0 + 1, nbuf1, sem1)
        return n_compute(g0 + 1, nbuf1, carry)

    carry0 = (jnp.int32(NUM_GRAPHS), zero16, (zero16,) * 8)
    lax.fori_loop(0, N_BLOCKS // 2, npair_body, carry0)

    pltpu.sync_copy(accn.at[pl.ds(0, _SEG_F)], out_n.at[w])
    pltpu.sync_copy(accc.at[pl.ds(0, _CNT_F)], out_c.at[w])


def _tc_edge_reduce(e_ref, o_ref):
    i = pl.program_id(0)
    x = e_ref[...]
    partial = jnp.sum(x.reshape(EDGE_DIM, EB_TC // 128, 128), axis=1)

    @pl.when(i == 0)
    def _():
        o_ref[...] = partial

    @pl.when(i > 0)
    def _():
        o_ref[...] += partial


def _tc_mlp(u_ref, np_ref, nc_ref, ep_ref, w1u_ref, w1n_ref, w1e_ref,
            b1_ref, w2_ref, b2_ref, o_ref):
    nagg = jnp.sum(np_ref[...], axis=0)
    cnt = jnp.sum(nc_ref[...], axis=0)[:, 0:1]
    cnt = jnp.maximum(cnt, 1.0)
    nagg = nagg / cnt
    esum = jnp.sum(ep_ref[...], axis=1)[None, :]
    emean = esum * (1.0 / N_EDGES)
    hp = jax.lax.Precision.HIGHEST
    h = (jnp.dot(u_ref[...], w1u_ref[...], precision=hp,
                 preferred_element_type=jnp.float32)
         + jnp.dot(nagg, w1n_ref[...], precision=hp,
                   preferred_element_type=jnp.float32)
         + jnp.dot(emean, w1e_ref[...], precision=hp,
                   preferred_element_type=jnp.float32)
         + b1_ref[...])
    h = jnp.maximum(h, 0.0)
    o_ref[...] = (jnp.dot(h, w2_ref[...], precision=hp,
                          preferred_element_type=jnp.float32)
                  + b2_ref[...])


def kernel(node_attr, edge_attr, u, batch, W1, b1, W2, b2):
    batch32 = batch.astype(jnp.int32)

    out_n, out_c = _sc_segsum(node_attr, batch32)

    edge_t = edge_attr.T
    ep = pl.pallas_call(
        _tc_edge_reduce,
        grid=(N_EDGES // EB_TC,),
        in_specs=[pl.BlockSpec((EDGE_DIM, EB_TC), lambda i: (0, i))],
        out_specs=pl.BlockSpec((EDGE_DIM, 128), lambda i: (0, 0)),
        out_shape=jax.ShapeDtypeStruct((EDGE_DIM, 128), jnp.float32),
    )(edge_t)

    np3 = out_n.reshape(NW, NUM_GRAPHS, NODE_DIM)
    nc3 = out_c.reshape(NW, NUM_GRAPHS, 16)

    w1u_t = W1[:, :NODE_DIM].T
    w1n_t = W1[:, NODE_DIM:2 * NODE_DIM].T
    w1e_t = W1[:, 2 * NODE_DIM:].T
    w2_t = W2.T
    b1r = b1.reshape(1, LATENT)
    b2r = b2.reshape(1, LATENT)

    return pl.pallas_call(
        _tc_mlp,
        out_shape=jax.ShapeDtypeStruct((NUM_GRAPHS, LATENT), jnp.float32),
    )(u, np3, nc3, ep, w1u_t, w1n_t, w1e_t, b1r, w2_t, b2r)


# speedup vs baseline: 13.8100x; 1.0016x over previous
import functools

import jax
import jax.numpy as jnp
from jax import lax
from jax.experimental import pallas as pl
from jax.experimental.pallas import tpu as pltpu
from jax.experimental.pallas import tpu_sc as plsc

N_NODES = 100000
N_EDGES = 3200000
NODE_DIM = 128
EDGE_DIM = 16
NUM_GRAPHS = 64
LATENT = 128

NW = 32

N_PER_W = 3128
N_LAST_W = N_NODES - 31 * N_PER_W
N_PAD_W = 3200
NB = 160
N_BLOCKS = N_PAD_W // NB

EB_TC = 320000

_SEG_F = NUM_GRAPHS * NODE_DIM
_CNT_F = NUM_GRAPHS * 16
_ACCN_F = (NUM_GRAPHS + 1) * NODE_DIM
_ACCC_F = (NUM_GRAPHS + 1) * 16


@functools.partial(
    pl.kernel,
    mesh=plsc.VectorSubcoreMesh(core_axis_name="c", subcore_axis_name="s"),
    compiler_params=pltpu.CompilerParams(needs_layout_passes=False),
    out_type=[
        jax.ShapeDtypeStruct((NW, _SEG_F), jnp.float32),
        jax.ShapeDtypeStruct((NW, _CNT_F), jnp.float32),
    ],
    scratch_types=[
        pltpu.VMEM((NB, NODE_DIM), jnp.float32),
        pltpu.VMEM((NB, NODE_DIM), jnp.float32),
        pltpu.VMEM((N_PAD_W,), jnp.int32),
        pltpu.VMEM((_ACCN_F,), jnp.float32),
        pltpu.VMEM((_ACCC_F,), jnp.float32),
        pltpu.SemaphoreType.DMA,
        pltpu.SemaphoreType.DMA,
        pltpu.SemaphoreType.DMA,
    ],
)
def _sc_segsum(node_hbm, batch_hbm, out_n, out_c,
               nbuf0, nbuf1, bbuf, accn, accc, sem0, sem1, semb):
    c = lax.axis_index("c")
    s = lax.axis_index("s")
    w = s * 2 + c

    zero16 = jnp.zeros((16,), jnp.float32)
    lane = lax.iota(jnp.int32, 16)
    ones16 = jnp.ones((16,), jnp.float32)
    pad16 = jnp.full((16,), NUM_GRAPHS, jnp.int32)

    nbase = w * N_PER_W
    rows_w = jnp.where(w == NW - 1, N_LAST_W, N_PER_W)
    rows_w = pl.multiple_of(rows_w, 8)

    for t in range(3024 // 16, N_PAD_W // 16):
        bbuf[pl.ds(t * 16, 16)] = pad16
    pltpu.async_copy(batch_hbm.at[pl.ds(nbase, rows_w)],
                     bbuf.at[pl.ds(0, rows_w)], semb)

    def nsize(g):
        return pl.multiple_of(jnp.clip(rows_w - g * NB, 0, NB), 8)

    def n_pair(g, buf):
        nrows = nsize(g)
        return (node_hbm.at[pl.ds(nbase + g * NB, nrows)],
                buf.at[pl.ds(0, nrows)], nrows)

    def n_start(g, buf, sem):
        src, dst, nrows = n_pair(g, buf)

        @pl.when(nrows > 0)
        def _():
            pltpu.async_copy(src, dst, sem)

    def n_wait(g, buf, sem):
        src, dst, nrows = n_pair(g, buf)

        @pl.when(nrows > 0)
        def _():
            pltpu.make_async_copy(src, dst, sem).wait()

    def flush_run(prev_sid, cnt, accs):
        idx0 = prev_sid * NODE_DIM + lane

        @pl.when(prev_sid < NUM_GRAPHS)
        def _():
            for j in range(8):
                plsc.addupdate_scatter(accn, [idx0 + j * 16], accs[j])
            plsc.addupdate_scatter(accc, [prev_sid * 16 + lane], cnt)

    def n_compute(g, buf, carry):
        def ngrp(t, carry):
            prev_sid, cnt, accs = carry
            sids = bbuf[pl.ds(g * NB + t * 16, 16)]
            for l in range(16):
                sid = sids[l]
                flush = sid != prev_sid
                row = t * 16 + l
                rowv = tuple(buf[row, pl.ds(j * 16, 16)] for j in range(8))

                @pl.when(flush)
                def _(prev_sid=prev_sid, cnt=cnt, accs=accs):
                    flush_run(prev_sid, cnt, accs)

                accs = tuple(jnp.where(flush, rowv[j], accs[j] + rowv[j])
                             for j in range(8))
                cnt = jnp.where(flush, ones16, cnt + ones16)
                prev_sid = sid
            return (prev_sid, cnt, accs)

        return lax.fori_loop(0, NB // 16, ngrp, carry)

    n_start(0, nbuf0, sem0)

    def zseg(j, _):
        accn[pl.ds(j * 16, 16)] = zero16
        return 0

    lax.fori_loop(0, _ACCN_F // 16, zseg, 0)

    def zcnt(j, _):
        accc[pl.ds(j * 16, 16)] = zero16
        return 0

    lax.fori_loop(0, _ACCC_F // 16, zcnt, 0)

    pltpu.make_async_copy(batch_hbm.at[pl.ds(nbase, rows_w)],
                          bbuf.at[pl.ds(0, rows_w)], semb).wait()

    def npair_body(p, carry):
        g0 = 2 * p
        n_start(g0 + 1, nbuf1, sem1)
        n_wait(g0, nbuf0, sem0)
        carry = n_compute(g0, nbuf0, carry)
        n_start(g0 + 2, nbuf0, sem0)
        n_wait(g0 + 1, nbuf1, sem1)
        return n_compute(g0 + 1, nbuf1, carry)

    carry0 = (jnp.int32(NUM_GRAPHS), zero16, (zero16,) * 8)
    lax.fori_loop(0, N_BLOCKS // 2, npair_body, carry0)

    pltpu.sync_copy(accn.at[pl.ds(0, _SEG_F)], out_n.at[w])
    pltpu.sync_copy(accc.at[pl.ds(0, _CNT_F)], out_c.at[w])


def _tc_edge_reduce(e_ref, o_ref):
    i = pl.program_id(0)
    x = e_ref[...]
    partial = jnp.sum(x.reshape(EDGE_DIM, EB_TC // 128, 128), axis=1)

    @pl.when(i == 0)
    def _():
        o_ref[...] = partial

    @pl.when(i > 0)
    def _():
        o_ref[...] += partial


def _tc_mlp(u_ref, np_ref, nc_ref, ep_ref, w1u_ref, w1n_ref, w1e_ref,
            b1_ref, w2_ref, b2_ref, o_ref):
    nagg = jnp.sum(np_ref[...], axis=0)
    cnt = jnp.sum(nc_ref[...], axis=0)[:, 0:1]
    cnt = jnp.maximum(cnt, 1.0)
    nagg = nagg / cnt
    esum = jnp.sum(ep_ref[...], axis=1)[None, :]
    emean = esum * (1.0 / N_EDGES)
    hp = jax.lax.Precision.HIGHEST
    h = (jnp.dot(u_ref[...], w1u_ref[...], precision=hp,
                 preferred_element_type=jnp.float32)
         + jnp.dot(nagg, w1n_ref[...], precision=hp,
                   preferred_element_type=jnp.float32)
         + jnp.dot(emean, w1e_ref[...], precision=hp,
                   preferred_element_type=jnp.float32)
         + b1_ref[...])
    h = jnp.maximum(h, 0.0)
    o_ref[...] = (jnp.dot(h, w2_ref[...], precision=hp,
                          preferred_element_type=jnp.float32)
                  + b2_ref[...])


def kernel(node_attr, edge_attr, u, batch, W1, b1, W2, b2):
    batch32 = batch.astype(jnp.int32)

    out_n, out_c = _sc_segsum(node_attr, batch32)

    edge_t = edge_attr.T
    ep = pl.pallas_call(
        _tc_edge_reduce,
        grid=(N_EDGES // EB_TC,),
        in_specs=[pl.BlockSpec((EDGE_DIM, EB_TC), lambda i: (0, i))],
        out_specs=pl.BlockSpec((EDGE_DIM, 128), lambda i: (0, 0)),
        out_shape=jax.ShapeDtypeStruct((EDGE_DIM, 128), jnp.float32),
    )(edge_t)

    np3 = out_n.reshape(NW, NUM_GRAPHS, NODE_DIM)
    nc3 = out_c.reshape(NW, NUM_GRAPHS, 16)

    w1u_t = W1[:, :NODE_DIM].T
    w1n_t = W1[:, NODE_DIM:2 * NODE_DIM].T
    w1e_t = W1[:, 2 * NODE_DIM:].T
    w2_t = W2.T
    b1r = b1.reshape(1, LATENT)
    b2r = b2.reshape(1, LATENT)

    return pl.pallas_call(
        _tc_mlp,
        out_shape=jax.ShapeDtypeStruct((NUM_GRAPHS, LATENT), jnp.float32),
    )(u, np3, nc3, ep, w1u_t, w1n_t, w1e_t, b1r, w2_t, b2r)
